# baseline (speedup 1.0000x reference)
import jax
import jax.numpy as jnp
from jax.experimental import pallas as pl
from jax.experimental.pallas import tpu as pltpu

_L, _B, _H, _S, _D = 8, 2, 8, 4096, 128
_F = _L * _B * _H
_C = 32
_FC = _F // _C


def _copy_patch(pos_ref, src_ref, new_ref, dst_ref, sem_bulk, sem_patch):
    for c in range(_C):
        pltpu.make_async_copy(
            src_ref.at[pl.ds(c * _FC, _FC)],
            dst_ref.at[pl.ds(c * _FC, _FC)],
            sem_bulk,
        ).start()
    for c in range(_C):
        pltpu.make_async_copy(
            src_ref.at[pl.ds(c * _FC, _FC)],
            dst_ref.at[pl.ds(c * _FC, _FC)],
            sem_bulk,
        ).wait()
    for f in range(_F):
        b = (f // _H) % _B
        pos = pos_ref[b]
        pltpu.make_async_copy(
            new_ref.at[f],
            dst_ref.at[f, pl.ds(pos, 1)],
            sem_patch,
        ).start()
    for f in range(_F):
        b = (f // _H) % _B
        pos = pos_ref[b]
        pltpu.make_async_copy(
            new_ref.at[f],
            dst_ref.at[f, pl.ds(pos, 1)],
            sem_patch,
        ).wait()


def _dma_body(pos_ref, k_ref, v_ref, nk_ref, nv_ref, ko_ref, vo_ref,
              sem_bulk, sem_patch):
    i = pl.program_id(0)

    @pl.when(i == 0)
    def _():
        _copy_patch(pos_ref, k_ref, nk_ref, ko_ref, sem_bulk, sem_patch)

    @pl.when(i == 1)
    def _():
        _copy_patch(pos_ref, v_ref, nv_ref, vo_ref, sem_bulk, sem_patch)


def kernel(k_cache, v_cache, new_k, new_v, attention_mask, position_ids):
    mask = attention_mask[:, None, None, :].astype(bool)

    kf = k_cache.reshape(_F, _S, _D)
    vf = v_cache.reshape(_F, _S, _D)
    nk = new_k.reshape(_F, 1, _D)
    nv = new_v.reshape(_F, 1, _D)
    pos = position_ids.reshape(_B)

    any_spec = pl.BlockSpec(memory_space=pl.ANY)
    grid_spec = pltpu.PrefetchScalarGridSpec(
        num_scalar_prefetch=1,
        grid=(2,),
        in_specs=[any_spec] * 4,
        out_specs=[any_spec] * 2,
        scratch_shapes=[pltpu.SemaphoreType.DMA, pltpu.SemaphoreType.DMA],
    )
    ko, vo = pl.pallas_call(
        _dma_body,
        grid_spec=grid_spec,
        out_shape=[jax.ShapeDtypeStruct((_F, _S, _D), k_cache.dtype)] * 2,
        compiler_params=pltpu.CompilerParams(
            dimension_semantics=("parallel",),
        ),
    )(pos, kf, vf, nk, nv)

    return (
        mask,
        ko.reshape(_L, _B, _H, _S, _D),
        vo.reshape(_L, _B, _H, _S, _D),
    )


# speedup vs baseline: 47.6719x; 47.6719x over previous
import jax
import jax.numpy as jnp
from jax.experimental import pallas as pl
from jax.experimental.pallas import tpu as pltpu

_L, _B, _H, _S, _D = 8, 2, 8, 4096, 128
_F = _L * _B * _H
_R = 1


def _scatter_body(pos_ref, k_ref, v_ref, nk_ref, nv_ref, ko_ref, vo_ref):
    i = pl.program_id(0)
    b = (i * _R // _H) % _B
    pos = pos_ref[b]
    sel = jax.lax.broadcasted_iota(jnp.int32, (1, _S, 1), 1) == pos
    ko_ref[...] = jnp.where(sel, nk_ref[...], k_ref[...])
    vo_ref[...] = jnp.where(sel, nv_ref[...], v_ref[...])


def kernel(k_cache, v_cache, new_k, new_v, attention_mask, position_ids):
    mask = attention_mask[:, None, None, :].astype(bool)

    kf = k_cache.reshape(_F, _S, _D)
    vf = v_cache.reshape(_F, _S, _D)
    nk = new_k.reshape(_F, 1, _D)
    nv = new_v.reshape(_F, 1, _D)
    pos = position_ids.reshape(_B)

    big = pl.BlockSpec((_R, _S, _D), lambda i, pos_ref: (i, 0, 0))
    row = pl.BlockSpec((_R, 1, _D), lambda i, pos_ref: (i, 0, 0))

    grid_spec = pltpu.PrefetchScalarGridSpec(
        num_scalar_prefetch=1,
        grid=(_F // _R,),
        in_specs=[big, big, row, row],
        out_specs=[big, big],
    )
    ko, vo = pl.pallas_call(
        _scatter_body,
        grid_spec=grid_spec,
        out_shape=[jax.ShapeDtypeStruct((_F, _S, _D), k_cache.dtype)] * 2,
        compiler_params=pltpu.CompilerParams(
            dimension_semantics=("parallel",),
            vmem_limit_bytes=48 * 1024 * 1024,
        ),
    )(pos, kf, vf, nk, nv)

    return (
        mask,
        ko.reshape(_L, _B, _H, _S, _D),
        vo.reshape(_L, _B, _H, _S, _D),
    )


# speedup vs baseline: 48.2330x; 1.0118x over previous
import jax
import jax.numpy as jnp
from jax.experimental import pallas as pl
from jax.experimental.pallas import tpu as pltpu

_L, _B, _H, _S, _D = 8, 2, 8, 4096, 128
_F = _L * _B * _H
_R = 2


def _scatter_body(pos_ref, k_ref, v_ref, nk_ref, nv_ref, am_ref,
                  ko_ref, vo_ref, m_ref):
    i = pl.program_id(0)
    b = (i * _R // _H) % _B
    pos = pos_ref[b]
    sel = jax.lax.broadcasted_iota(jnp.int32, (1, _S, 1), 1) == pos
    f0 = pl.multiple_of(i * _R, _R)
    ko_ref[...] = jnp.where(sel, nk_ref[pl.ds(f0, _R)], k_ref[...])
    vo_ref[...] = jnp.where(sel, nv_ref[pl.ds(f0, _R)], v_ref[...])
    m_ref[...] = (am_ref[...] != 0).reshape(_B, 1, 1, _S)


def kernel(k_cache, v_cache, new_k, new_v, attention_mask, position_ids):
    kf = k_cache.reshape(_F, _S, _D)
    vf = v_cache.reshape(_F, _S, _D)
    nk = new_k.reshape(_F, 1, _D)
    nv = new_v.reshape(_F, 1, _D)
    pos = position_ids.reshape(_B)

    big = pl.BlockSpec((_R, _S, _D), lambda i, pos_ref: (i, 0, 0))
    whole_rows = pl.BlockSpec((_F, 1, _D), lambda i, pos_ref: (0, 0, 0))
    whole_am = pl.BlockSpec((_B, _S), lambda i, pos_ref: (0, 0))
    whole_mask = pl.BlockSpec((_B, 1, 1, _S), lambda i, pos_ref: (0, 0, 0, 0))

    grid_spec = pltpu.PrefetchScalarGridSpec(
        num_scalar_prefetch=1,
        grid=(_F // _R,),
        in_specs=[big, big, whole_rows, whole_rows, whole_am],
        out_specs=[big, big, whole_mask],
    )
    ko, vo, mask = pl.pallas_call(
        _scatter_body,
        grid_spec=grid_spec,
        out_shape=[
            jax.ShapeDtypeStruct((_F, _S, _D), k_cache.dtype),
            jax.ShapeDtypeStruct((_F, _S, _D), v_cache.dtype),
            jax.ShapeDtypeStruct((_B, 1, 1, _S), jnp.bool_),
        ],
        compiler_params=pltpu.CompilerParams(
            dimension_semantics=("parallel",),
            vmem_limit_bytes=48 * 1024 * 1024,
        ),
    )(pos, kf, vf, nk, nv, attention_mask)

    return (
        mask,
        ko.reshape(_L, _B, _H, _S, _D),
        vo.reshape(_L, _B, _H, _S, _D),
    )
